# baseline (speedup 1.0000x reference)
import jax
import jax.numpy as jnp
from jax import lax
from jax.experimental import pallas as pl
from jax.experimental.pallas import tpu as pltpu
from jax.experimental.pallas import tpu_sc as plsc

_NW = 32
_WORDS_PER_W = 320000
_CHUNK = 32000
_N_CHUNKS = 10
_SLOTS = 3
_W = 2


def _sc_copy(in_hbm, out_hbm, buf, in_sems, out_sems):
    wid = lax.axis_index("s") * 2 + lax.axis_index("c")
    base = pl.multiple_of(wid * _WORDS_PER_W, 8)

    def in_copy(c):
        return pltpu.make_async_copy(
            in_hbm.at[pl.ds(base + c * _CHUNK, _CHUNK)],
            buf.at[pl.ds((c % _SLOTS) * _CHUNK, _CHUNK)],
            in_sems.at[c % _SLOTS])

    def out_copy(c):
        return pltpu.make_async_copy(
            buf.at[pl.ds((c % _SLOTS) * _CHUNK, _CHUNK)],
            out_hbm.at[pl.ds(base + c * _CHUNK, _CHUNK)],
            out_sems.at[c % _SLOTS])

    for c in range(_W):
        in_copy(c).start()
    for c in range(_N_CHUNKS):
        in_copy(c).wait()
        out_copy(c).start()
        n = c + _W
        if n < _N_CHUNKS:
            if n >= _SLOTS:
                out_copy(n - _SLOTS).wait()
            in_copy(n).start()
    for c in range(_N_CHUNKS - _SLOTS, _N_CHUNKS):
        out_copy(c).wait()


def kernel(samples, sample_rate):
    x = samples.reshape(-1)
    mesh = plsc.VectorSubcoreMesh(core_axis_name="c", subcore_axis_name="s")
    run = pl.kernel(
        _sc_copy,
        out_type=jax.ShapeDtypeStruct(x.shape, x.dtype),
        mesh=mesh,
        scratch_types=[
            pltpu.VMEM((_SLOTS * _CHUNK,), x.dtype),
            pltpu.SemaphoreType.DMA((_SLOTS,)),
            pltpu.SemaphoreType.DMA((_SLOTS,)),
        ],
    )
    return run(x).reshape(samples.shape)


# speedup vs baseline: 1.8492x; 1.8492x over previous
import jax
import jax.numpy as jnp
from jax.experimental import pallas as pl
from jax.experimental.pallas import tpu as pltpu

_LANES = 128
_N_CHUNKS = 64
_SLOTS = 32
_W = 16


def _copy_kernel(chunk_rows, in_hbm, out_hbm, buf, in_sems, out_sems):
    def in_copy(c):
        return pltpu.make_async_copy(
            in_hbm.at[pl.ds(c * chunk_rows, chunk_rows)],
            buf.at[c % _SLOTS],
            in_sems.at[c % _SLOTS])

    def out_copy(c):
        return pltpu.make_async_copy(
            buf.at[c % _SLOTS],
            out_hbm.at[pl.ds(c * chunk_rows, chunk_rows)],
            out_sems.at[c % _SLOTS])

    for c in range(_W):
        in_copy(c).start()
    for c in range(_N_CHUNKS):
        in_copy(c).wait()
        out_copy(c).start()
        n = c + _W
        if n < _N_CHUNKS:
            if n >= _SLOTS:
                out_copy(n - _SLOTS).wait()
            in_copy(n).start()
    for c in range(max(0, _N_CHUNKS - _SLOTS), _N_CHUNKS):
        out_copy(c).wait()


def kernel(samples, sample_rate):
    rows = samples.size // _LANES
    chunk_rows = rows // _N_CHUNKS
    x = samples.reshape(rows, _LANES)
    out = pl.pallas_call(
        lambda *a: _copy_kernel(chunk_rows, *a),
        out_shape=jax.ShapeDtypeStruct(x.shape, x.dtype),
        in_specs=[pl.BlockSpec(memory_space=pltpu.MemorySpace.HBM)],
        out_specs=pl.BlockSpec(memory_space=pltpu.MemorySpace.HBM),
        scratch_shapes=[
            pltpu.VMEM((_SLOTS, chunk_rows, _LANES), x.dtype),
            pltpu.SemaphoreType.DMA((_SLOTS,)),
            pltpu.SemaphoreType.DMA((_SLOTS,)),
        ],
    )(x)
    return out.reshape(samples.shape)


# speedup vs baseline: 1.8843x; 1.0190x over previous
import jax
import jax.numpy as jnp
from jax.experimental import pallas as pl
from jax.experimental.pallas import tpu as pltpu

_LANES = 128
_N_CHUNKS = 16
_SLOTS = 8
_W = 4


def _copy_kernel(chunk_rows, in_hbm, out_hbm, buf, in_sems, out_sems):
    def in_copy(c):
        return pltpu.make_async_copy(
            in_hbm.at[pl.ds(c * chunk_rows, chunk_rows)],
            buf.at[c % _SLOTS],
            in_sems.at[c % _SLOTS])

    def out_copy(c):
        return pltpu.make_async_copy(
            buf.at[c % _SLOTS],
            out_hbm.at[pl.ds(c * chunk_rows, chunk_rows)],
            out_sems.at[c % _SLOTS])

    for c in range(_W):
        in_copy(c).start()
    for c in range(_N_CHUNKS):
        in_copy(c).wait()
        out_copy(c).start()
        n = c + _W
        if n < _N_CHUNKS:
            if n >= _SLOTS:
                out_copy(n - _SLOTS).wait()
            in_copy(n).start()
    for c in range(max(0, _N_CHUNKS - _SLOTS), _N_CHUNKS):
        out_copy(c).wait()


def kernel(samples, sample_rate):
    rows = samples.size // _LANES
    chunk_rows = rows // _N_CHUNKS
    x = samples.reshape(rows, _LANES)
    out = pl.pallas_call(
        lambda *a: _copy_kernel(chunk_rows, *a),
        out_shape=jax.ShapeDtypeStruct(x.shape, x.dtype),
        in_specs=[pl.BlockSpec(memory_space=pltpu.MemorySpace.HBM)],
        out_specs=pl.BlockSpec(memory_space=pltpu.MemorySpace.HBM),
        scratch_shapes=[
            pltpu.VMEM((_SLOTS, chunk_rows, _LANES), x.dtype),
            pltpu.SemaphoreType.DMA((_SLOTS,)),
            pltpu.SemaphoreType.DMA((_SLOTS,)),
        ],
    )(x)
    return out.reshape(samples.shape)
